# baseline (speedup 1.0000x reference)
import jax
import jax.numpy as jnp
from jax import lax
from jax.experimental import pallas as pl
from jax.experimental.pallas import tpu as pltpu
from jax.experimental.pallas import tpu_sc as plsc

_N = 10000
_E = 320000
_NC = 2
_NS = 16
_NW = _NC * _NS
_CH = 128
_EROWS = 2560
_GPT = _EROWS // _NW // 8
_RB = 624
_ZR = 16
_DW = 16
_NACC = _N + 16


_R0 = 160
_R1 = 160 - _R0
_C1BASE = _NS * _R0


def _make_sc_agg(d):
  mesh = plsc.VectorSubcoreMesh(core_axis_name="c", subcore_axis_name="s",
                                num_cores=_NC, num_subcores=_NS)
  out_type = [jax.ShapeDtypeStruct((_NC, _N, d), jnp.float32)]
  scratch = [
      pltpu.VMEM((8, _CH), jnp.int32),
      pltpu.VMEM((8, _CH), jnp.int32),
      pltpu.VMEM((2, _CH, d), jnp.float32),
      pltpu.VMEM((_ZR, d), jnp.float32),
      pltpu.VMEM_SHARED((_NACC, d), jnp.float32),
      pltpu.SemaphoreType.DMA,
      pltpu.SemaphoreType.DMA,
  ]

  def body(g_hbm, srcm, dstm, out_hbm, sidx, didx, rows2, zbuf, acc,
           semg, sems):
    cid = lax.axis_index("c")
    sid = lax.axis_index("s")
    tbase = jnp.where(cid == 0, sid * _R0, _C1BASE + sid * _R1)
    ngroups = jnp.where(cid == 0, _R0 // 8, _R1 // 8)

    def zrow(i, _):
      def zlane(j, _):
        zbuf[i, pl.ds(j * 16, 16)] = jnp.zeros((16,), jnp.float32)
        return 0
      return lax.fori_loop(0, d // 16, zlane, 0)
    lax.fori_loop(0, _ZR, zrow, 0)
    rbase = sid * _RB
    def zcopy(k, _):
      pltpu.sync_copy(zbuf, acc.at[pl.ds(rbase + k * _ZR, _ZR)])
      return 0
    lax.fori_loop(0, _RB // _ZR, zcopy, 0)
    @pl.when(sid == _NS - 1)
    def _():
      pltpu.sync_copy(zbuf, acc.at[pl.ds(_NS * _RB, _N - _NS * _RB)])

    plsc.subcore_barrier()

    def group(k, _):
      gbase = pl.multiple_of((tbase + k * 8) * 1, 8)
      pltpu.sync_copy(srcm.at[pl.ds(gbase, 8)], sidx)
      pltpu.sync_copy(dstm.at[pl.ds(gbase, 8)], didx)
      scat = {}
      gat = pltpu.async_copy(g_hbm.at[sidx.at[0]], rows2.at[0], semg)
      for j in range(8):
        gat.wait()
        scat[j] = pltpu.async_copy(rows2.at[j % 2], acc.at[didx.at[j]],
                                   sems, add=True)
        if j < 7:
          if j >= 1:
            scat[j - 1].wait()
          gat = pltpu.async_copy(g_hbm.at[sidx.at[j + 1]],
                                 rows2.at[(j + 1) % 2], semg)
      scat[6].wait()
      scat[7].wait()
      return 0
    lax.fori_loop(0, ngroups, group, 0)

    plsc.subcore_barrier()
    pltpu.sync_copy(acc.at[pl.ds(rbase, _RB)],
                    out_hbm.at[cid, pl.ds(rbase, _RB)])
    @pl.when(sid == _NS - 1)
    def _():
      pltpu.sync_copy(acc.at[pl.ds(_NS * _RB, _N - _NS * _RB)],
                      out_hbm.at[cid, pl.ds(_NS * _RB, _N - _NS * _RB)])

  return pl.kernel(body, out_type=out_type, mesh=mesh, scratch_types=scratch)


def _make_sc_deg():
  mesh = plsc.VectorSubcoreMesh(core_axis_name="c", subcore_axis_name="s",
                                num_cores=_NC, num_subcores=_NS)
  out_type = [jax.ShapeDtypeStruct((_NC, _N, 128), jnp.float32)]
  scratch = [
      pltpu.VMEM((8, _CH), jnp.int32),
      pltpu.VMEM((_CH, 128), jnp.float32),
      pltpu.VMEM((_ZR, 128), jnp.float32),
      pltpu.VMEM_SHARED((_NACC, 128), jnp.float32),
  ]

  def body(dstm, deg_hbm, didx, ones, zdeg, degacc):
    cid = lax.axis_index("c")
    sid = lax.axis_index("s")
    wid = sid * _NC + cid
    def onesrow(i, _):
      def onelane(j, _):
        ones[i, pl.ds(j * 16, 16)] = jnp.ones((16,), jnp.float32)
        return 0
      return lax.fori_loop(0, 8, onelane, 0)
    lax.fori_loop(0, _CH, onesrow, 0)
    def zdrow(i, _):
      def zlane(j, _):
        zdeg[i, pl.ds(j * 16, 16)] = jnp.zeros((16,), jnp.float32)
        return 0
      return lax.fori_loop(0, 8, zlane, 0)
    lax.fori_loop(0, _ZR, zdrow, 0)
    rbase = sid * _RB
    def zdcopy(k, _):
      pltpu.sync_copy(zdeg, degacc.at[pl.ds(rbase + k * _ZR, _ZR)])
      return 0
    lax.fori_loop(0, _RB // _ZR, zdcopy, 0)
    @pl.when(sid == _NS - 1)
    def _():
      pltpu.sync_copy(zdeg, degacc.at[pl.ds(_NS * _RB, _N - _NS * _RB)])
    plsc.subcore_barrier()
    def group(k, _):
      gbase = pl.multiple_of((wid * _GPT + k) * 8, 8)
      pltpu.sync_copy(dstm.at[pl.ds(gbase, 8)], didx)
      for j in range(8):
        pltpu.sync_copy(ones, degacc.at[didx.at[j]], add=True)
      return 0
    lax.fori_loop(0, _GPT, group, 0)
    plsc.subcore_barrier()
    pltpu.sync_copy(degacc.at[pl.ds(rbase, _RB)],
                    deg_hbm.at[cid, pl.ds(rbase, _RB)])
    @pl.when(sid == _NS - 1)
    def _():
      pltpu.sync_copy(degacc.at[pl.ds(_NS * _RB, _N - _NS * _RB)],
                      deg_hbm.at[cid, pl.ds(_NS * _RB, _N - _NS * _RB)])

  return pl.kernel(body, out_type=out_type, mesh=mesh, scratch_types=scratch)


_AGG128 = _make_sc_agg(128)
_SC_DEG = _make_sc_deg()

_R = 1000
_EPAD = _EROWS * _CH


def _make_tc(pre, relu, din, da, dout, dnext):
  grid = (_N // _R,)
  row = lambda w: pl.BlockSpec((_R, w), lambda i: (i, 0))
  full = lambda a, b_: pl.BlockSpec((a, b_), lambda i: (0, 0))
  in_specs = [row(da), row(da), row(1), row(1), row(din)]
  if pre:
    in_specs.append(full(da, dout))
  in_specs.append(full(din, dout))
  in_specs.append(full(1, dout))
  if dnext:
    in_specs.append(full(dout, dnext))
  out_shape = [jax.ShapeDtypeStruct((_N, dout), jnp.float32)]
  out_specs = [row(dout)]
  if dnext:
    out_shape.append(jax.ShapeDtypeStruct((_N, dnext), jnp.float32))
    out_specs.append(row(dnext))

  def kfn(*refs):
    p0, p1, dg0, dg1, h = (r[...] for r in refs[:5])
    i = 5
    if pre:
      Wl = refs[i][...]; i += 1
    Wr = refs[i][...]; i += 1
    b = refs[i][...]; i += 1
    Wn = None
    if dnext:
      Wn = refs[i][...]; i += 1
    out_ref = refs[i]; i += 1
    gn_ref = refs[i] if dnext else None
    recip = 1.0 / jnp.maximum(dg0 + dg1, 1.0)
    mean = (p0 + p1) * recip
    if pre:
      acc = (jnp.dot(mean, Wl, preferred_element_type=jnp.float32)
             + jnp.dot(h, Wr, preferred_element_type=jnp.float32) + b)
    else:
      acc = mean + jnp.dot(h, Wr, preferred_element_type=jnp.float32) + b
    if relu:
      acc = jnp.maximum(acc, 0.0)
    out_ref[...] = acc
    if dnext:
      gn_ref[...] = jnp.dot(acc, Wn, preferred_element_type=jnp.float32)

  return pl.pallas_call(kfn, grid=grid, in_specs=in_specs,
                        out_specs=out_specs, out_shape=out_shape)


_TC1 = _make_tc(pre=True, relu=True, din=128, da=128, dout=256, dnext=128)
_TC2 = _make_tc(pre=False, relu=True, din=256, da=128, dout=128, dnext=128)
_TC3 = _make_tc(pre=False, relu=True, din=128, da=128, dout=128, dnext=None)
_TC4 = _make_tc(pre=True, relu=False, din=128, da=128, dout=128, dnext=None)


def kernel(x, edge_index, W1l, b1, W1r, W2l, b2, W2r, W3l, b3, W3r,
           W4l, b4, W4r):
  src = edge_index[0]
  dst = edge_index[1]
  srcm = jnp.concatenate(
      [src, jnp.zeros((_EPAD - _E,), jnp.int32)]).reshape(_EROWS, _CH)
  dstm = jnp.concatenate(
      [dst, jnp.full((_EPAD - _E,), _N, jnp.int32)]).reshape(_EROWS, _CH)
  W3l_p = jnp.pad(W3l, ((0, 0), (0, 64)))
  W3r_p = jnp.pad(W3r, ((0, 0), (0, 64)))
  b3_p = jnp.pad(b3, (0, 64))
  W4l_p = jnp.pad(W4l, ((0, 64), (0, 0)))
  W4r_p = jnp.pad(W4r, ((0, 64), (0, 0)))

  (degs,) = _SC_DEG(dstm)
  dg0 = degs[0, :, 0:1]
  dg1 = degs[1, :, 0:1]
  (p1,) = _AGG128(x, srcm, dstm)

  h1, g2 = _TC1(p1[0], p1[1], dg0, dg1, x, W1l, W1r, b1.reshape(1, -1), W2l)
  (p2,) = _AGG128(g2, srcm, dstm)
  h2, g3 = _TC2(p2[0], p2[1], dg0, dg1, h1, W2r, b2.reshape(1, -1), W3l_p)
  (p3,) = _AGG128(g3, srcm, dstm)
  (h3,) = _TC3(p3[0], p3[1], dg0, dg1, h2, W3r_p, b3_p.reshape(1, -1))
  (p4,) = _AGG128(h3, srcm, dstm)
  (out,) = _TC4(p4[0], p4[1], dg0, dg1, h3, W4l_p, W4r_p, b4.reshape(1, -1))
  return out


# speedup vs baseline: 1.2575x; 1.2575x over previous
import jax
import jax.numpy as jnp
from jax import lax
from jax.experimental import pallas as pl
from jax.experimental.pallas import tpu as pltpu
from jax.experimental.pallas import tpu_sc as plsc

_N = 10000
_E = 320000
_NC = 2
_NS = 16
_NW = _NC * _NS
_CH = 128
_EROWS = 2560
_GPT = _EROWS // _NW // 8
_RB = 624
_ZR = 16
_DW = 16
_NACC = _N + 16


_R0 = 104
_R1 = 160 - _R0
_C1BASE = _NS * _R0


def _make_sc_agg(d):
  mesh = plsc.VectorSubcoreMesh(core_axis_name="c", subcore_axis_name="s",
                                num_cores=_NC, num_subcores=_NS)
  out_type = [jax.ShapeDtypeStruct((_NC, _N, d), jnp.float32)]
  scratch = [
      pltpu.VMEM((8, _CH), jnp.int32),
      pltpu.VMEM((8, _CH), jnp.int32),
      pltpu.VMEM((2, _CH, d), jnp.float32),
      pltpu.VMEM((_ZR, d), jnp.float32),
      pltpu.VMEM_SHARED((_NACC, d), jnp.float32),
      pltpu.SemaphoreType.DMA,
      pltpu.SemaphoreType.DMA,
  ]

  def body(g_hbm, srcm, dstm, out_hbm, sidx, didx, rows2, zbuf, acc,
           semg, sems):
    cid = lax.axis_index("c")
    sid = lax.axis_index("s")
    tbase = jnp.where(cid == 0, sid * _R0, _C1BASE + sid * _R1)
    ngroups = jnp.where(cid == 0, _R0 // 8, _R1 // 8)

    def zrow(i, _):
      def zlane(j, _):
        zbuf[i, pl.ds(j * 16, 16)] = jnp.zeros((16,), jnp.float32)
        return 0
      return lax.fori_loop(0, d // 16, zlane, 0)
    lax.fori_loop(0, _ZR, zrow, 0)
    rbase = sid * _RB
    def zcopy(k, _):
      pltpu.sync_copy(zbuf, acc.at[pl.ds(rbase + k * _ZR, _ZR)])
      return 0
    lax.fori_loop(0, _RB // _ZR, zcopy, 0)
    @pl.when(sid == _NS - 1)
    def _():
      pltpu.sync_copy(zbuf, acc.at[pl.ds(_NS * _RB, _N - _NS * _RB)])

    plsc.subcore_barrier()

    def group(k, _):
      gbase = pl.multiple_of((tbase + k * 8) * 1, 8)
      pltpu.sync_copy(srcm.at[pl.ds(gbase, 8)], sidx)
      pltpu.sync_copy(dstm.at[pl.ds(gbase, 8)], didx)
      scat = {}
      gat = pltpu.async_copy(g_hbm.at[sidx.at[0]], rows2.at[0], semg)
      for j in range(8):
        gat.wait()
        scat[j] = pltpu.async_copy(rows2.at[j % 2], acc.at[didx.at[j]],
                                   sems, add=True)
        if j < 7:
          if j >= 1:
            scat[j - 1].wait()
          gat = pltpu.async_copy(g_hbm.at[sidx.at[j + 1]],
                                 rows2.at[(j + 1) % 2], semg)
      scat[6].wait()
      scat[7].wait()
      return 0
    lax.fori_loop(0, ngroups, group, 0)

    plsc.subcore_barrier()
    pltpu.sync_copy(acc.at[pl.ds(rbase, _RB)],
                    out_hbm.at[cid, pl.ds(rbase, _RB)])
    @pl.when(sid == _NS - 1)
    def _():
      pltpu.sync_copy(acc.at[pl.ds(_NS * _RB, _N - _NS * _RB)],
                      out_hbm.at[cid, pl.ds(_NS * _RB, _N - _NS * _RB)])

  return pl.kernel(body, out_type=out_type, mesh=mesh, scratch_types=scratch)


def _make_sc_deg():
  mesh = plsc.VectorSubcoreMesh(core_axis_name="c", subcore_axis_name="s",
                                num_cores=_NC, num_subcores=_NS)
  out_type = [jax.ShapeDtypeStruct((_NC, _N, 128), jnp.float32)]
  scratch = [
      pltpu.VMEM((8, _CH), jnp.int32),
      pltpu.VMEM((_CH, 128), jnp.float32),
      pltpu.VMEM((_ZR, 128), jnp.float32),
      pltpu.VMEM_SHARED((_NACC, 128), jnp.float32),
  ]

  def body(dstm, deg_hbm, didx, ones, zdeg, degacc):
    cid = lax.axis_index("c")
    sid = lax.axis_index("s")
    wid = sid * _NC + cid
    def onesrow(i, _):
      def onelane(j, _):
        ones[i, pl.ds(j * 16, 16)] = jnp.ones((16,), jnp.float32)
        return 0
      return lax.fori_loop(0, 8, onelane, 0)
    lax.fori_loop(0, _CH, onesrow, 0)
    def zdrow(i, _):
      def zlane(j, _):
        zdeg[i, pl.ds(j * 16, 16)] = jnp.zeros((16,), jnp.float32)
        return 0
      return lax.fori_loop(0, 8, zlane, 0)
    lax.fori_loop(0, _ZR, zdrow, 0)
    rbase = sid * _RB
    def zdcopy(k, _):
      pltpu.sync_copy(zdeg, degacc.at[pl.ds(rbase + k * _ZR, _ZR)])
      return 0
    lax.fori_loop(0, _RB // _ZR, zdcopy, 0)
    @pl.when(sid == _NS - 1)
    def _():
      pltpu.sync_copy(zdeg, degacc.at[pl.ds(_NS * _RB, _N - _NS * _RB)])
    plsc.subcore_barrier()
    def group(k, _):
      gbase = pl.multiple_of((wid * _GPT + k) * 8, 8)
      pltpu.sync_copy(dstm.at[pl.ds(gbase, 8)], didx)
      for j in range(8):
        pltpu.sync_copy(ones, degacc.at[didx.at[j]], add=True)
      return 0
    lax.fori_loop(0, _GPT, group, 0)
    plsc.subcore_barrier()
    pltpu.sync_copy(degacc.at[pl.ds(rbase, _RB)],
                    deg_hbm.at[cid, pl.ds(rbase, _RB)])
    @pl.when(sid == _NS - 1)
    def _():
      pltpu.sync_copy(degacc.at[pl.ds(_NS * _RB, _N - _NS * _RB)],
                      deg_hbm.at[cid, pl.ds(_NS * _RB, _N - _NS * _RB)])

  return pl.kernel(body, out_type=out_type, mesh=mesh, scratch_types=scratch)


_AGG128 = _make_sc_agg(128)
_SC_DEG = _make_sc_deg()

_R = 1000
_EPAD = _EROWS * _CH


def _make_tc(pre, relu, din, da, dout, dnext):
  grid = (_N // _R,)
  row = lambda w: pl.BlockSpec((_R, w), lambda i: (i, 0))
  full = lambda a, b_: pl.BlockSpec((a, b_), lambda i: (0, 0))
  in_specs = [row(da), row(da), row(1), row(1), row(din)]
  if pre:
    in_specs.append(full(da, dout))
  in_specs.append(full(din, dout))
  in_specs.append(full(1, dout))
  if dnext:
    in_specs.append(full(dout, dnext))
  out_shape = [jax.ShapeDtypeStruct((_N, dout), jnp.float32)]
  out_specs = [row(dout)]
  if dnext:
    out_shape.append(jax.ShapeDtypeStruct((_N, dnext), jnp.float32))
    out_specs.append(row(dnext))

  def kfn(*refs):
    p0, p1, dg0, dg1, h = (r[...] for r in refs[:5])
    i = 5
    if pre:
      Wl = refs[i][...]; i += 1
    Wr = refs[i][...]; i += 1
    b = refs[i][...]; i += 1
    Wn = None
    if dnext:
      Wn = refs[i][...]; i += 1
    out_ref = refs[i]; i += 1
    gn_ref = refs[i] if dnext else None
    recip = 1.0 / jnp.maximum(dg0 + dg1, 1.0)
    mean = (p0 + p1) * recip
    if pre:
      acc = (jnp.dot(mean, Wl, preferred_element_type=jnp.float32)
             + jnp.dot(h, Wr, preferred_element_type=jnp.float32) + b)
    else:
      acc = mean + jnp.dot(h, Wr, preferred_element_type=jnp.float32) + b
    if relu:
      acc = jnp.maximum(acc, 0.0)
    out_ref[...] = acc
    if dnext:
      gn_ref[...] = jnp.dot(acc, Wn, preferred_element_type=jnp.float32)

  return pl.pallas_call(kfn, grid=grid, in_specs=in_specs,
                        out_specs=out_specs, out_shape=out_shape)


_TC1 = _make_tc(pre=True, relu=True, din=128, da=128, dout=256, dnext=128)
_TC2 = _make_tc(pre=False, relu=True, din=256, da=128, dout=128, dnext=128)
_TC3 = _make_tc(pre=False, relu=True, din=128, da=128, dout=128, dnext=None)
_TC4 = _make_tc(pre=True, relu=False, din=128, da=128, dout=128, dnext=None)


def kernel(x, edge_index, W1l, b1, W1r, W2l, b2, W2r, W3l, b3, W3r,
           W4l, b4, W4r):
  src = edge_index[0]
  dst = edge_index[1]
  srcm = jnp.concatenate(
      [src, jnp.zeros((_EPAD - _E,), jnp.int32)]).reshape(_EROWS, _CH)
  dstm = jnp.concatenate(
      [dst, jnp.full((_EPAD - _E,), _N, jnp.int32)]).reshape(_EROWS, _CH)
  W3l_p = jnp.pad(W3l, ((0, 0), (0, 64)))
  W3r_p = jnp.pad(W3r, ((0, 0), (0, 64)))
  b3_p = jnp.pad(b3, (0, 64))
  W4l_p = jnp.pad(W4l, ((0, 64), (0, 0)))
  W4r_p = jnp.pad(W4r, ((0, 64), (0, 0)))

  (degs,) = _SC_DEG(dstm)
  dg0 = degs[0, :, 0:1]
  dg1 = degs[1, :, 0:1]
  (p1,) = _AGG128(x, srcm, dstm)

  h1, g2 = _TC1(p1[0], p1[1], dg0, dg1, x, W1l, W1r, b1.reshape(1, -1), W2l)
  (p2,) = _AGG128(g2, srcm, dstm)
  h2, g3 = _TC2(p2[0], p2[1], dg0, dg1, h1, W2r, b2.reshape(1, -1), W3l_p)
  (p3,) = _AGG128(g3, srcm, dstm)
  (h3,) = _TC3(p3[0], p3[1], dg0, dg1, h2, W3r_p, b3_p.reshape(1, -1))
  (p4,) = _AGG128(h3, srcm, dstm)
  (out,) = _TC4(p4[0], p4[1], dg0, dg1, h3, W4l_p, W4r_p, b4.reshape(1, -1))
  return out


# speedup vs baseline: 3.1615x; 2.5141x over previous
import jax
import jax.numpy as jnp
from jax import lax
from jax.experimental import pallas as pl
from jax.experimental.pallas import tpu as pltpu
from jax.experimental.pallas import tpu_sc as plsc

_N = 10000
_E = 320000
_NC = 2
_NS = 16
_NW = _NC * _NS
_CH = 128
_EROWS = 2560
_GPT = _EROWS // _NW // 8
_RB = 624
_ZR = 16
_DW = 16
_NACC = _N + 16


_R0 = 80
_R1 = 160 - _R0
_C1BASE = _NS * _R0


def _make_sc_agg(d):
  mesh = plsc.VectorSubcoreMesh(core_axis_name="c", subcore_axis_name="s",
                                num_cores=_NC, num_subcores=_NS)
  out_type = [jax.ShapeDtypeStruct((_NC, _N, d), jnp.float32)]
  scratch = [
      pltpu.VMEM((8, _CH), jnp.int32),
      pltpu.VMEM((8, _CH), jnp.int32),
      pltpu.VMEM((2, _CH, d), jnp.float32),
      pltpu.VMEM((_ZR, d), jnp.float32),
      pltpu.VMEM_SHARED((_NACC, d), jnp.float32),
      pltpu.SemaphoreType.DMA,
      pltpu.SemaphoreType.DMA,
  ]

  def body(g_hbm, srcm, dstm, out_hbm, sidx, didx, rows2, zbuf, acc,
           semg, sems):
    cid = lax.axis_index("c")
    sid = lax.axis_index("s")
    tbase = jnp.where(cid == 0, sid * _R0, _C1BASE + sid * _R1)
    ngroups = jnp.where(cid == 0, _R0 // 8, _R1 // 8)

    def zrow(i, _):
      def zlane(j, _):
        zbuf[i, pl.ds(j * 16, 16)] = jnp.zeros((16,), jnp.float32)
        return 0
      return lax.fori_loop(0, d // 16, zlane, 0)
    lax.fori_loop(0, _ZR, zrow, 0)
    rbase = sid * _RB
    def zcopy(k, _):
      pltpu.sync_copy(zbuf, acc.at[pl.ds(rbase + k * _ZR, _ZR)])
      return 0
    lax.fori_loop(0, _RB // _ZR, zcopy, 0)
    @pl.when(sid == _NS - 1)
    def _():
      pltpu.sync_copy(zbuf, acc.at[pl.ds(_NS * _RB, _N - _NS * _RB)])

    plsc.subcore_barrier()

    def group(k, _):
      gbase = pl.multiple_of((tbase + k * 8) * 1, 8)
      pltpu.sync_copy(srcm.at[pl.ds(gbase, 8)], sidx)
      pltpu.sync_copy(dstm.at[pl.ds(gbase, 8)], didx)
      scat = {}
      gat = pltpu.async_copy(g_hbm.at[sidx.at[0]], rows2.at[0], semg)
      for j in range(8):
        gat.wait()
        scat[j] = pltpu.async_copy(rows2.at[j % 2], acc.at[didx.at[j]],
                                   sems, add=True)
        if j < 7:
          if j >= 1:
            scat[j - 1].wait()
          gat = pltpu.async_copy(g_hbm.at[sidx.at[j + 1]],
                                 rows2.at[(j + 1) % 2], semg)
      scat[6].wait()
      scat[7].wait()
      return 0
    lax.fori_loop(0, ngroups, group, 0)

    plsc.subcore_barrier()
    pltpu.sync_copy(acc.at[pl.ds(rbase, _RB)],
                    out_hbm.at[cid, pl.ds(rbase, _RB)])
    @pl.when(sid == _NS - 1)
    def _():
      pltpu.sync_copy(acc.at[pl.ds(_NS * _RB, _N - _NS * _RB)],
                      out_hbm.at[cid, pl.ds(_NS * _RB, _N - _NS * _RB)])

  return pl.kernel(body, out_type=out_type, mesh=mesh, scratch_types=scratch)


def _make_sc_deg():
  mesh = plsc.VectorSubcoreMesh(core_axis_name="c", subcore_axis_name="s",
                                num_cores=_NC, num_subcores=_NS)
  out_type = [jax.ShapeDtypeStruct((_NC, _N, 128), jnp.float32)]
  scratch = [
      pltpu.VMEM((8, _CH), jnp.int32),
      pltpu.VMEM((_CH, 128), jnp.float32),
      pltpu.VMEM((_ZR, 128), jnp.float32),
      pltpu.VMEM_SHARED((_NACC, 128), jnp.float32),
  ]

  def body(dstm, deg_hbm, didx, ones, zdeg, degacc):
    cid = lax.axis_index("c")
    sid = lax.axis_index("s")
    wid = sid * _NC + cid
    def onesrow(i, _):
      def onelane(j, _):
        ones[i, pl.ds(j * 16, 16)] = jnp.ones((16,), jnp.float32)
        return 0
      return lax.fori_loop(0, 8, onelane, 0)
    lax.fori_loop(0, _CH, onesrow, 0)
    def zdrow(i, _):
      def zlane(j, _):
        zdeg[i, pl.ds(j * 16, 16)] = jnp.zeros((16,), jnp.float32)
        return 0
      return lax.fori_loop(0, 8, zlane, 0)
    lax.fori_loop(0, _ZR, zdrow, 0)
    rbase = sid * _RB
    def zdcopy(k, _):
      pltpu.sync_copy(zdeg, degacc.at[pl.ds(rbase + k * _ZR, _ZR)])
      return 0
    lax.fori_loop(0, _RB // _ZR, zdcopy, 0)
    @pl.when(sid == _NS - 1)
    def _():
      pltpu.sync_copy(zdeg, degacc.at[pl.ds(_NS * _RB, _N - _NS * _RB)])
    plsc.subcore_barrier()
    def group(k, _):
      gbase = pl.multiple_of((wid * _GPT + k) * 8, 8)
      pltpu.sync_copy(dstm.at[pl.ds(gbase, 8)], didx)
      for j in range(8):
        pltpu.sync_copy(ones, degacc.at[didx.at[j]], add=True)
      return 0
    lax.fori_loop(0, _GPT, group, 0)
    plsc.subcore_barrier()
    pltpu.sync_copy(degacc.at[pl.ds(rbase, _RB)],
                    deg_hbm.at[cid, pl.ds(rbase, _RB)])
    @pl.when(sid == _NS - 1)
    def _():
      pltpu.sync_copy(degacc.at[pl.ds(_NS * _RB, _N - _NS * _RB)],
                      deg_hbm.at[cid, pl.ds(_NS * _RB, _N - _NS * _RB)])

  return pl.kernel(body, out_type=out_type, mesh=mesh, scratch_types=scratch)


_AGG128 = _make_sc_agg(128)
_SC_DEG = _make_sc_deg()

_R = 1000
_EPAD = _EROWS * _CH


def _make_tc(pre, relu, din, da, dout, dnext):
  grid = (_N // _R,)
  row = lambda w: pl.BlockSpec((_R, w), lambda i: (i, 0))
  full = lambda a, b_: pl.BlockSpec((a, b_), lambda i: (0, 0))
  in_specs = [row(da), row(da), row(1), row(1), row(din)]
  if pre:
    in_specs.append(full(da, dout))
  in_specs.append(full(din, dout))
  in_specs.append(full(1, dout))
  if dnext:
    in_specs.append(full(dout, dnext))
  out_shape = [jax.ShapeDtypeStruct((_N, dout), jnp.float32)]
  out_specs = [row(dout)]
  if dnext:
    out_shape.append(jax.ShapeDtypeStruct((_N, dnext), jnp.float32))
    out_specs.append(row(dnext))

  def kfn(*refs):
    p0, p1, dg0, dg1, h = (r[...] for r in refs[:5])
    i = 5
    if pre:
      Wl = refs[i][...]; i += 1
    Wr = refs[i][...]; i += 1
    b = refs[i][...]; i += 1
    Wn = None
    if dnext:
      Wn = refs[i][...]; i += 1
    out_ref = refs[i]; i += 1
    gn_ref = refs[i] if dnext else None
    recip = 1.0 / jnp.maximum(dg0 + dg1, 1.0)
    mean = (p0 + p1) * recip
    if pre:
      acc = (jnp.dot(mean, Wl, preferred_element_type=jnp.float32)
             + jnp.dot(h, Wr, preferred_element_type=jnp.float32) + b)
    else:
      acc = mean + jnp.dot(h, Wr, preferred_element_type=jnp.float32) + b
    if relu:
      acc = jnp.maximum(acc, 0.0)
    out_ref[...] = acc
    if dnext:
      gn_ref[...] = jnp.dot(acc, Wn, preferred_element_type=jnp.float32)

  return pl.pallas_call(kfn, grid=grid, in_specs=in_specs,
                        out_specs=out_specs, out_shape=out_shape)


_TC1 = _make_tc(pre=True, relu=True, din=128, da=128, dout=256, dnext=128)
_TC2 = _make_tc(pre=False, relu=True, din=256, da=128, dout=128, dnext=128)
_TC3 = _make_tc(pre=False, relu=True, din=128, da=128, dout=128, dnext=None)
_TC4 = _make_tc(pre=True, relu=False, din=128, da=128, dout=128, dnext=None)


def kernel(x, edge_index, W1l, b1, W1r, W2l, b2, W2r, W3l, b3, W3r,
           W4l, b4, W4r):
  src = edge_index[0]
  dst = edge_index[1]
  pad_src = (jnp.arange(_EPAD - _E, dtype=jnp.int32) * 131) % _N
  srcm = jnp.concatenate([src, pad_src]).reshape(_EROWS, _CH)
  dstm = jnp.concatenate(
      [dst, jnp.full((_EPAD - _E,), _N, jnp.int32)]).reshape(_EROWS, _CH)
  W3l_p = jnp.pad(W3l, ((0, 0), (0, 64)))
  W3r_p = jnp.pad(W3r, ((0, 0), (0, 64)))
  b3_p = jnp.pad(b3, (0, 64))
  W4l_p = jnp.pad(W4l, ((0, 64), (0, 0)))
  W4r_p = jnp.pad(W4r, ((0, 64), (0, 0)))

  (degs,) = _SC_DEG(dstm)
  dg0 = degs[0, :, 0:1]
  dg1 = degs[1, :, 0:1]
  (p1,) = _AGG128(x, srcm, dstm)

  h1, g2 = _TC1(p1[0], p1[1], dg0, dg1, x, W1l, W1r, b1.reshape(1, -1), W2l)
  (p2,) = _AGG128(g2, srcm, dstm)
  h2, g3 = _TC2(p2[0], p2[1], dg0, dg1, h1, W2r, b2.reshape(1, -1), W3l_p)
  (p3,) = _AGG128(g3, srcm, dstm)
  (h3,) = _TC3(p3[0], p3[1], dg0, dg1, h2, W3r_p, b3_p.reshape(1, -1))
  (p4,) = _AGG128(h3, srcm, dstm)
  (out,) = _TC4(p4[0], p4[1], dg0, dg1, h3, W4l_p, W4r_p, b4.reshape(1, -1))
  return out


# speedup vs baseline: 3.2378x; 1.0241x over previous
import jax
import jax.numpy as jnp
from jax import lax
from jax.experimental import pallas as pl
from jax.experimental.pallas import tpu as pltpu
from jax.experimental.pallas import tpu_sc as plsc

_N = 10000
_E = 320000
_NC = 2
_NS = 16
_NW = _NC * _NS
_CH = 128
_EROWS = 2560
_GPT = _EROWS // _NW // 8
_RB = 624
_ZR = 16
_DW = 16
_NACC = _N + 16


_R0 = 80
_R1 = 160 - _R0
_C1BASE = _NS * _R0


def _make_sc_agg(d):
  mesh = plsc.VectorSubcoreMesh(core_axis_name="c", subcore_axis_name="s",
                                num_cores=_NC, num_subcores=_NS)
  out_type = [jax.ShapeDtypeStruct((_NC, _N, d), jnp.float32)]
  scratch = [
      pltpu.VMEM((2, 8, _CH), jnp.int32),
      pltpu.VMEM((2, 8, _CH), jnp.int32),
      pltpu.VMEM((2, _CH, d), jnp.float32),
      pltpu.VMEM((_ZR, d), jnp.float32),
      pltpu.VMEM_SHARED((_NACC, d), jnp.float32),
      pltpu.SemaphoreType.DMA,
      pltpu.SemaphoreType.DMA,
  ]

  def body(g_hbm, srcm, dstm, out_hbm, sidx, didx, rows2, zbuf, acc,
           semg, sems):
    cid = lax.axis_index("c")
    sid = lax.axis_index("s")
    tbase = jnp.where(cid == 0, sid * _R0, _C1BASE + sid * _R1)
    ngroups = jnp.where(cid == 0, _R0 // 8, _R1 // 8)

    g0 = pl.multiple_of(tbase + 0, 8)
    pltpu.sync_copy(srcm.at[pl.ds(g0, 8)], sidx.at[0])
    pltpu.sync_copy(dstm.at[pl.ds(g0, 8)], didx.at[0])

    def zrow(i, _):
      def zlane(j, _):
        zbuf[i, pl.ds(j * 16, 16)] = jnp.zeros((16,), jnp.float32)
        return 0
      return lax.fori_loop(0, d // 16, zlane, 0)
    lax.fori_loop(0, _ZR, zrow, 0)
    rbase = sid * _RB
    def zcopy(k, _):
      pltpu.sync_copy(zbuf, acc.at[pl.ds(rbase + k * _ZR, _ZR)])
      return 0
    lax.fori_loop(0, _RB // _ZR, zcopy, 0)
    @pl.when(sid == _NS - 1)
    def _():
      pltpu.sync_copy(zbuf, acc.at[pl.ds(_NS * _RB, _N - _NS * _RB)])

    plsc.subcore_barrier()

    def group(k, _):
      cur = k % 2
      scat = {}
      gat = pltpu.async_copy(g_hbm.at[sidx.at[cur, 0]], rows2.at[0], semg)
      for j in range(8):
        gat.wait()
        scat[j] = pltpu.async_copy(rows2.at[j % 2], acc.at[didx.at[cur, j]],
                                   sems, add=True)
        if j < 7:
          if j >= 1:
            scat[j - 1].wait()
          gat = pltpu.async_copy(g_hbm.at[sidx.at[cur, j + 1]],
                                 rows2.at[(j + 1) % 2], semg)
      @pl.when(k + 1 < ngroups)
      def _():
        nbase = pl.multiple_of(tbase + (k + 1) * 8, 8)
        pltpu.sync_copy(srcm.at[pl.ds(nbase, 8)], sidx.at[1 - cur])
        pltpu.sync_copy(dstm.at[pl.ds(nbase, 8)], didx.at[1 - cur])
      scat[6].wait()
      scat[7].wait()
      return 0
    lax.fori_loop(0, ngroups, group, 0)

    plsc.subcore_barrier()
    pltpu.sync_copy(acc.at[pl.ds(rbase, _RB)],
                    out_hbm.at[cid, pl.ds(rbase, _RB)])
    @pl.when(sid == _NS - 1)
    def _():
      pltpu.sync_copy(acc.at[pl.ds(_NS * _RB, _N - _NS * _RB)],
                      out_hbm.at[cid, pl.ds(_NS * _RB, _N - _NS * _RB)])

  return pl.kernel(body, out_type=out_type, mesh=mesh, scratch_types=scratch)


def _make_sc_deg():
  mesh = plsc.VectorSubcoreMesh(core_axis_name="c", subcore_axis_name="s",
                                num_cores=_NC, num_subcores=_NS)
  out_type = [jax.ShapeDtypeStruct((_NC, _N, 128), jnp.float32)]
  scratch = [
      pltpu.VMEM((8, _CH), jnp.int32),
      pltpu.VMEM((_CH, 128), jnp.float32),
      pltpu.VMEM((_ZR, 128), jnp.float32),
      pltpu.VMEM_SHARED((_NACC, 128), jnp.float32),
  ]

  def body(dstm, deg_hbm, didx, ones, zdeg, degacc):
    cid = lax.axis_index("c")
    sid = lax.axis_index("s")
    wid = sid * _NC + cid
    def onesrow(i, _):
      def onelane(j, _):
        ones[i, pl.ds(j * 16, 16)] = jnp.ones((16,), jnp.float32)
        return 0
      return lax.fori_loop(0, 8, onelane, 0)
    lax.fori_loop(0, _CH, onesrow, 0)
    def zdrow(i, _):
      def zlane(j, _):
        zdeg[i, pl.ds(j * 16, 16)] = jnp.zeros((16,), jnp.float32)
        return 0
      return lax.fori_loop(0, 8, zlane, 0)
    lax.fori_loop(0, _ZR, zdrow, 0)
    rbase = sid * _RB
    def zdcopy(k, _):
      pltpu.sync_copy(zdeg, degacc.at[pl.ds(rbase + k * _ZR, _ZR)])
      return 0
    lax.fori_loop(0, _RB // _ZR, zdcopy, 0)
    @pl.when(sid == _NS - 1)
    def _():
      pltpu.sync_copy(zdeg, degacc.at[pl.ds(_NS * _RB, _N - _NS * _RB)])
    plsc.subcore_barrier()
    def group(k, _):
      gbase = pl.multiple_of((wid * _GPT + k) * 8, 8)
      pltpu.sync_copy(dstm.at[pl.ds(gbase, 8)], didx)
      for j in range(8):
        pltpu.sync_copy(ones, degacc.at[didx.at[j]], add=True)
      return 0
    lax.fori_loop(0, _GPT, group, 0)
    plsc.subcore_barrier()
    pltpu.sync_copy(degacc.at[pl.ds(rbase, _RB)],
                    deg_hbm.at[cid, pl.ds(rbase, _RB)])
    @pl.when(sid == _NS - 1)
    def _():
      pltpu.sync_copy(degacc.at[pl.ds(_NS * _RB, _N - _NS * _RB)],
                      deg_hbm.at[cid, pl.ds(_NS * _RB, _N - _NS * _RB)])

  return pl.kernel(body, out_type=out_type, mesh=mesh, scratch_types=scratch)


_AGG128 = _make_sc_agg(128)
_SC_DEG = _make_sc_deg()

_R = 1000
_EPAD = _EROWS * _CH


def _make_tc(pre, relu, din, da, dout, dnext):
  grid = (_N // _R,)
  row = lambda w: pl.BlockSpec((_R, w), lambda i: (i, 0))
  full = lambda a, b_: pl.BlockSpec((a, b_), lambda i: (0, 0))
  in_specs = [row(da), row(da), row(1), row(1), row(din)]
  if pre:
    in_specs.append(full(da, dout))
  in_specs.append(full(din, dout))
  in_specs.append(full(1, dout))
  if dnext:
    in_specs.append(full(dout, dnext))
  out_shape = [jax.ShapeDtypeStruct((_N, dout), jnp.float32)]
  out_specs = [row(dout)]
  if dnext:
    out_shape.append(jax.ShapeDtypeStruct((_N, dnext), jnp.float32))
    out_specs.append(row(dnext))

  def kfn(*refs):
    p0, p1, dg0, dg1, h = (r[...] for r in refs[:5])
    i = 5
    if pre:
      Wl = refs[i][...]; i += 1
    Wr = refs[i][...]; i += 1
    b = refs[i][...]; i += 1
    Wn = None
    if dnext:
      Wn = refs[i][...]; i += 1
    out_ref = refs[i]; i += 1
    gn_ref = refs[i] if dnext else None
    recip = 1.0 / jnp.maximum(dg0 + dg1, 1.0)
    mean = (p0 + p1) * recip
    if pre:
      acc = (jnp.dot(mean, Wl, preferred_element_type=jnp.float32)
             + jnp.dot(h, Wr, preferred_element_type=jnp.float32) + b)
    else:
      acc = mean + jnp.dot(h, Wr, preferred_element_type=jnp.float32) + b
    if relu:
      acc = jnp.maximum(acc, 0.0)
    out_ref[...] = acc
    if dnext:
      gn_ref[...] = jnp.dot(acc, Wn, preferred_element_type=jnp.float32)

  return pl.pallas_call(kfn, grid=grid, in_specs=in_specs,
                        out_specs=out_specs, out_shape=out_shape)


_TC1 = _make_tc(pre=True, relu=True, din=128, da=128, dout=256, dnext=128)
_TC2 = _make_tc(pre=False, relu=True, din=256, da=128, dout=128, dnext=128)
_TC3 = _make_tc(pre=False, relu=True, din=128, da=128, dout=128, dnext=None)
_TC4 = _make_tc(pre=True, relu=False, din=128, da=128, dout=128, dnext=None)


def kernel(x, edge_index, W1l, b1, W1r, W2l, b2, W2r, W3l, b3, W3r,
           W4l, b4, W4r):
  src = edge_index[0]
  dst = edge_index[1]
  pad_src = (jnp.arange(_EPAD - _E, dtype=jnp.int32) * 131) % _N
  srcm = jnp.concatenate([src, pad_src]).reshape(_EROWS, _CH)
  dstm = jnp.concatenate(
      [dst, jnp.full((_EPAD - _E,), _N, jnp.int32)]).reshape(_EROWS, _CH)
  W3l_p = jnp.pad(W3l, ((0, 0), (0, 64)))
  W3r_p = jnp.pad(W3r, ((0, 0), (0, 64)))
  b3_p = jnp.pad(b3, (0, 64))
  W4l_p = jnp.pad(W4l, ((0, 64), (0, 0)))
  W4r_p = jnp.pad(W4r, ((0, 64), (0, 0)))

  (degs,) = _SC_DEG(dstm)
  dg0 = degs[0, :, 0:1]
  dg1 = degs[1, :, 0:1]
  (p1,) = _AGG128(x, srcm, dstm)

  h1, g2 = _TC1(p1[0], p1[1], dg0, dg1, x, W1l, W1r, b1.reshape(1, -1), W2l)
  (p2,) = _AGG128(g2, srcm, dstm)
  h2, g3 = _TC2(p2[0], p2[1], dg0, dg1, h1, W2r, b2.reshape(1, -1), W3l_p)
  (p3,) = _AGG128(g3, srcm, dstm)
  (h3,) = _TC3(p3[0], p3[1], dg0, dg1, h2, W3r_p, b3_p.reshape(1, -1))
  (p4,) = _AGG128(h3, srcm, dstm)
  (out,) = _TC4(p4[0], p4[1], dg0, dg1, h3, W4l_p, W4r_p, b4.reshape(1, -1))
  return out


# speedup vs baseline: 3.3148x; 1.0238x over previous
import jax
import jax.numpy as jnp
from jax import lax
from jax.experimental import pallas as pl
from jax.experimental.pallas import tpu as pltpu
from jax.experimental.pallas import tpu_sc as plsc

_N = 10000
_E = 320000
_NC = 2
_NS = 16
_NW = _NC * _NS
_CH = 128
_EROWS = 2560
_GPT = _EROWS // _NW // 8
_RB = 624
_ZR = 16
_DW = 16
_NACC = _N + 16


_R0 = 80
_R1 = 160 - _R0
_C1BASE = _NS * _R0


def _make_sc_agg(d):
  mesh = plsc.VectorSubcoreMesh(core_axis_name="c", subcore_axis_name="s",
                                num_cores=_NC, num_subcores=_NS)
  out_type = [jax.ShapeDtypeStruct((_NC, _N, d), jnp.float32)]
  scratch = [
      pltpu.VMEM((2, 16, _CH), jnp.int32),
      pltpu.VMEM((2, 16, _CH), jnp.int32),
      pltpu.VMEM((2, _CH, d), jnp.float32),
      pltpu.VMEM((_ZR, d), jnp.float32),
      pltpu.VMEM_SHARED((_NACC, d), jnp.float32),
      pltpu.SemaphoreType.DMA,
      pltpu.SemaphoreType.DMA,
  ]

  def body(g_hbm, srcm, dstm, out_hbm, sidx, didx, rows2, zbuf, acc,
           semg, sems):
    cid = lax.axis_index("c")
    sid = lax.axis_index("s")
    tbase = jnp.where(cid == 0, sid * _R0, _C1BASE + sid * _R1)
    ngroups = jnp.where(cid == 0, _R0 // 16, _R1 // 16)

    g0 = pl.multiple_of(tbase + 0, 8)
    pltpu.sync_copy(srcm.at[pl.ds(g0, 16)], sidx.at[0])
    pltpu.sync_copy(dstm.at[pl.ds(g0, 16)], didx.at[0])

    def zrow(i, _):
      def zlane(j, _):
        zbuf[i, pl.ds(j * 16, 16)] = jnp.zeros((16,), jnp.float32)
        return 0
      return lax.fori_loop(0, d // 16, zlane, 0)
    lax.fori_loop(0, _ZR, zrow, 0)
    rbase = sid * _RB
    def zcopy(k, _):
      pltpu.sync_copy(zbuf, acc.at[pl.ds(rbase + k * _ZR, _ZR)])
      return 0
    lax.fori_loop(0, _RB // _ZR, zcopy, 0)
    @pl.when(sid == _NS - 1)
    def _():
      pltpu.sync_copy(zbuf, acc.at[pl.ds(_NS * _RB, _N - _NS * _RB)])

    plsc.subcore_barrier()

    def group(k, _):
      cur = k % 2
      scat = {}
      gat = pltpu.async_copy(g_hbm.at[sidx.at[cur, 0]], rows2.at[0], semg)
      for j in range(16):
        gat.wait()
        scat[j] = pltpu.async_copy(rows2.at[j % 2], acc.at[didx.at[cur, j]],
                                   sems, add=True)
        if j < 15:
          if j >= 1:
            scat[j - 1].wait()
          gat = pltpu.async_copy(g_hbm.at[sidx.at[cur, j + 1]],
                                 rows2.at[(j + 1) % 2], semg)
      @pl.when(k + 1 < ngroups)
      def _():
        nbase = pl.multiple_of(tbase + (k + 1) * 16, 8)
        pltpu.sync_copy(srcm.at[pl.ds(nbase, 16)], sidx.at[1 - cur])
        pltpu.sync_copy(dstm.at[pl.ds(nbase, 16)], didx.at[1 - cur])
      scat[14].wait()
      scat[15].wait()
      return 0
    lax.fori_loop(0, ngroups, group, 0)

    plsc.subcore_barrier()
    pltpu.sync_copy(acc.at[pl.ds(rbase, _RB)],
                    out_hbm.at[cid, pl.ds(rbase, _RB)])
    @pl.when(sid == _NS - 1)
    def _():
      pltpu.sync_copy(acc.at[pl.ds(_NS * _RB, _N - _NS * _RB)],
                      out_hbm.at[cid, pl.ds(_NS * _RB, _N - _NS * _RB)])

  return pl.kernel(body, out_type=out_type, mesh=mesh, scratch_types=scratch)


def _make_sc_deg():
  mesh = plsc.VectorSubcoreMesh(core_axis_name="c", subcore_axis_name="s",
                                num_cores=_NC, num_subcores=_NS)
  out_type = [jax.ShapeDtypeStruct((_NC, _N, 128), jnp.float32)]
  scratch = [
      pltpu.VMEM((8, _CH), jnp.int32),
      pltpu.VMEM((_CH, 128), jnp.float32),
      pltpu.VMEM((_ZR, 128), jnp.float32),
      pltpu.VMEM_SHARED((_NACC, 128), jnp.float32),
  ]

  def body(dstm, deg_hbm, didx, ones, zdeg, degacc):
    cid = lax.axis_index("c")
    sid = lax.axis_index("s")
    wid = sid * _NC + cid
    def onesrow(i, _):
      def onelane(j, _):
        ones[i, pl.ds(j * 16, 16)] = jnp.ones((16,), jnp.float32)
        return 0
      return lax.fori_loop(0, 8, onelane, 0)
    lax.fori_loop(0, _CH, onesrow, 0)
    def zdrow(i, _):
      def zlane(j, _):
        zdeg[i, pl.ds(j * 16, 16)] = jnp.zeros((16,), jnp.float32)
        return 0
      return lax.fori_loop(0, 8, zlane, 0)
    lax.fori_loop(0, _ZR, zdrow, 0)
    rbase = sid * _RB
    def zdcopy(k, _):
      pltpu.sync_copy(zdeg, degacc.at[pl.ds(rbase + k * _ZR, _ZR)])
      return 0
    lax.fori_loop(0, _RB // _ZR, zdcopy, 0)
    @pl.when(sid == _NS - 1)
    def _():
      pltpu.sync_copy(zdeg, degacc.at[pl.ds(_NS * _RB, _N - _NS * _RB)])
    plsc.subcore_barrier()
    def group(k, _):
      gbase = pl.multiple_of((wid * _GPT + k) * 8, 8)
      pltpu.sync_copy(dstm.at[pl.ds(gbase, 8)], didx)
      for j in range(8):
        pltpu.sync_copy(ones, degacc.at[didx.at[j]], add=True)
      return 0
    lax.fori_loop(0, _GPT, group, 0)
    plsc.subcore_barrier()
    pltpu.sync_copy(degacc.at[pl.ds(rbase, _RB)],
                    deg_hbm.at[cid, pl.ds(rbase, _RB)])
    @pl.when(sid == _NS - 1)
    def _():
      pltpu.sync_copy(degacc.at[pl.ds(_NS * _RB, _N - _NS * _RB)],
                      deg_hbm.at[cid, pl.ds(_NS * _RB, _N - _NS * _RB)])

  return pl.kernel(body, out_type=out_type, mesh=mesh, scratch_types=scratch)


_AGG128 = _make_sc_agg(128)
_SC_DEG = _make_sc_deg()

_R = 1000
_EPAD = _EROWS * _CH


def _make_tc(pre, relu, din, da, dout, dnext):
  grid = (_N // _R,)
  row = lambda w: pl.BlockSpec((_R, w), lambda i: (i, 0))
  full = lambda a, b_: pl.BlockSpec((a, b_), lambda i: (0, 0))
  in_specs = [row(da), row(da), row(1), row(1), row(din)]
  if pre:
    in_specs.append(full(da, dout))
  in_specs.append(full(din, dout))
  in_specs.append(full(1, dout))
  if dnext:
    in_specs.append(full(dout, dnext))
  out_shape = [jax.ShapeDtypeStruct((_N, dout), jnp.float32)]
  out_specs = [row(dout)]
  if dnext:
    out_shape.append(jax.ShapeDtypeStruct((_N, dnext), jnp.float32))
    out_specs.append(row(dnext))

  def kfn(*refs):
    p0, p1, dg0, dg1, h = (r[...] for r in refs[:5])
    i = 5
    if pre:
      Wl = refs[i][...]; i += 1
    Wr = refs[i][...]; i += 1
    b = refs[i][...]; i += 1
    Wn = None
    if dnext:
      Wn = refs[i][...]; i += 1
    out_ref = refs[i]; i += 1
    gn_ref = refs[i] if dnext else None
    recip = 1.0 / jnp.maximum(dg0 + dg1, 1.0)
    mean = (p0 + p1) * recip
    if pre:
      acc = (jnp.dot(mean, Wl, preferred_element_type=jnp.float32)
             + jnp.dot(h, Wr, preferred_element_type=jnp.float32) + b)
    else:
      acc = mean + jnp.dot(h, Wr, preferred_element_type=jnp.float32) + b
    if relu:
      acc = jnp.maximum(acc, 0.0)
    out_ref[...] = acc
    if dnext:
      gn_ref[...] = jnp.dot(acc, Wn, preferred_element_type=jnp.float32)

  return pl.pallas_call(kfn, grid=grid, in_specs=in_specs,
                        out_specs=out_specs, out_shape=out_shape)


_TC1 = _make_tc(pre=True, relu=True, din=128, da=128, dout=256, dnext=128)
_TC2 = _make_tc(pre=False, relu=True, din=256, da=128, dout=128, dnext=128)
_TC3 = _make_tc(pre=False, relu=True, din=128, da=128, dout=128, dnext=None)
_TC4 = _make_tc(pre=True, relu=False, din=128, da=128, dout=128, dnext=None)


def kernel(x, edge_index, W1l, b1, W1r, W2l, b2, W2r, W3l, b3, W3r,
           W4l, b4, W4r):
  src = edge_index[0]
  dst = edge_index[1]
  pad_src = (jnp.arange(_EPAD - _E, dtype=jnp.int32) * 131) % _N
  srcm = jnp.concatenate([src, pad_src]).reshape(_EROWS, _CH)
  dstm = jnp.concatenate(
      [dst, jnp.full((_EPAD - _E,), _N, jnp.int32)]).reshape(_EROWS, _CH)
  W3l_p = jnp.pad(W3l, ((0, 0), (0, 64)))
  W3r_p = jnp.pad(W3r, ((0, 0), (0, 64)))
  b3_p = jnp.pad(b3, (0, 64))
  W4l_p = jnp.pad(W4l, ((0, 64), (0, 0)))
  W4r_p = jnp.pad(W4r, ((0, 64), (0, 0)))

  (degs,) = _SC_DEG(dstm)
  dg0 = degs[0, :, 0:1]
  dg1 = degs[1, :, 0:1]
  (p1,) = _AGG128(x, srcm, dstm)

  h1, g2 = _TC1(p1[0], p1[1], dg0, dg1, x, W1l, W1r, b1.reshape(1, -1), W2l)
  (p2,) = _AGG128(g2, srcm, dstm)
  h2, g3 = _TC2(p2[0], p2[1], dg0, dg1, h1, W2r, b2.reshape(1, -1), W3l_p)
  (p3,) = _AGG128(g3, srcm, dstm)
  (h3,) = _TC3(p3[0], p3[1], dg0, dg1, h2, W3r_p, b3_p.reshape(1, -1))
  (p4,) = _AGG128(h3, srcm, dstm)
  (out,) = _TC4(p4[0], p4[1], dg0, dg1, h3, W4l_p, W4r_p, b4.reshape(1, -1))
  return out


# speedup vs baseline: 3.3199x; 1.0015x over previous
import jax
import jax.numpy as jnp
from jax import lax
from jax.experimental import pallas as pl
from jax.experimental.pallas import tpu as pltpu
from jax.experimental.pallas import tpu_sc as plsc

_N = 10000
_E = 320000
_NC = 2
_NS = 16
_NW = _NC * _NS
_CH = 128
_EROWS = 2560
_GPT = _EROWS // _NW // 8
_RB = 624
_ZR = 16
_DW = 16
_NACC = _N + 16


_R0 = 80
_R1 = 160 - _R0
_C1BASE = _NS * _R0


def _make_sc_agg(d, with_deg=False):
  mesh = plsc.VectorSubcoreMesh(core_axis_name="c", subcore_axis_name="s",
                                num_cores=_NC, num_subcores=_NS)
  out_type = [jax.ShapeDtypeStruct((_NC, _N, d), jnp.float32)]
  if with_deg:
    out_type.append(jax.ShapeDtypeStruct((_NC, _N, d), jnp.float32))
  scratch = [
      pltpu.VMEM((2, 16, _CH), jnp.int32),
      pltpu.VMEM((2, 16, _CH), jnp.int32),
      pltpu.VMEM((2, _CH, d), jnp.float32),
      pltpu.VMEM((_ZR, d), jnp.float32),
      pltpu.VMEM_SHARED((_NACC, d), jnp.float32),
      pltpu.SemaphoreType.DMA,
      pltpu.SemaphoreType.DMA,
  ]

  def body(g_hbm, srcm, dstm, out_hbm, *rest):
    if with_deg:
      deg_hbm, sidx, didx, rows2, zbuf, acc, semg, sems = rest
    else:
      sidx, didx, rows2, zbuf, acc, semg, sems = rest
    cid = lax.axis_index("c")
    sid = lax.axis_index("s")
    tbase = jnp.where(cid == 0, sid * _R0, _C1BASE + sid * _R1)
    ngroups = jnp.where(cid == 0, _R0 // 16, _R1 // 16)

    g0 = pl.multiple_of(tbase + 0, 8)
    pltpu.sync_copy(srcm.at[pl.ds(g0, 16)], sidx.at[0])
    pltpu.sync_copy(dstm.at[pl.ds(g0, 16)], didx.at[0])

    def zrow(i, _):
      def zlane(j, _):
        zbuf[i, pl.ds(j * 16, 16)] = jnp.zeros((16,), jnp.float32)
        return 0
      return lax.fori_loop(0, d // 16, zlane, 0)
    lax.fori_loop(0, _ZR, zrow, 0)
    rbase = sid * _RB
    def zcopy(k, _):
      pltpu.sync_copy(zbuf, acc.at[pl.ds(rbase + k * _ZR, _ZR)])
      return 0
    lax.fori_loop(0, _RB // _ZR, zcopy, 0)
    @pl.when(sid == _NS - 1)
    def _():
      pltpu.sync_copy(zbuf, acc.at[pl.ds(_NS * _RB, _N - _NS * _RB)])

    plsc.subcore_barrier()

    if with_deg:
      def onesrow(i, _):
        def onelane(j, _):
          rows2[0, i, pl.ds(j * 16, 16)] = jnp.ones((16,), jnp.float32)
          return 0
        return lax.fori_loop(0, d // 16, onelane, 0)
      lax.fori_loop(0, _CH, onesrow, 0)

      def dgroup(k, _):
        cur = k % 2
        for j in range(16):
          pltpu.sync_copy(rows2.at[0], acc.at[didx.at[cur, j]], add=True)
        @pl.when(k + 1 < ngroups)
        def _():
          nbase = pl.multiple_of(tbase + (k + 1) * 16, 8)
          pltpu.sync_copy(dstm.at[pl.ds(nbase, 16)], didx.at[1 - cur])
        return 0
      lax.fori_loop(0, ngroups, dgroup, 0)

      plsc.subcore_barrier()
      pltpu.sync_copy(acc.at[pl.ds(rbase, _RB)],
                      deg_hbm.at[cid, pl.ds(rbase, _RB)])
      @pl.when(sid == _NS - 1)
      def _():
        pltpu.sync_copy(acc.at[pl.ds(_NS * _RB, _N - _NS * _RB)],
                        deg_hbm.at[cid, pl.ds(_NS * _RB, _N - _NS * _RB)])
      def rz(i, _):
        def rzl(j, _):
          rows2[0, i, pl.ds(j * 16, 16)] = jnp.zeros((16,), jnp.float32)
          return 0
        return lax.fori_loop(0, d // 16, rzl, 0)
      lax.fori_loop(0, _ZR, rz, 0)
      def rzcopy(k, _):
        pltpu.sync_copy(rows2.at[0, pl.ds(0, _ZR)],
                        acc.at[pl.ds(rbase + k * _ZR, _ZR)])
        return 0
      lax.fori_loop(0, _RB // _ZR, rzcopy, 0)
      @pl.when(sid == _NS - 1)
      def _():
        pltpu.sync_copy(rows2.at[0, pl.ds(0, _ZR)],
                        acc.at[pl.ds(_NS * _RB, _N - _NS * _RB)])
      pltpu.sync_copy(dstm.at[pl.ds(g0, 16)], didx.at[0])
      plsc.subcore_barrier()

    def group(k, _):
      cur = k % 2
      scat = {}
      gat = pltpu.async_copy(g_hbm.at[sidx.at[cur, 0]], rows2.at[0], semg)
      for j in range(16):
        gat.wait()
        scat[j] = pltpu.async_copy(rows2.at[j % 2], acc.at[didx.at[cur, j]],
                                   sems, add=True)
        if j < 15:
          if j >= 1:
            scat[j - 1].wait()
          gat = pltpu.async_copy(g_hbm.at[sidx.at[cur, j + 1]],
                                 rows2.at[(j + 1) % 2], semg)
      @pl.when(k + 1 < ngroups)
      def _():
        nbase = pl.multiple_of(tbase + (k + 1) * 16, 8)
        pltpu.sync_copy(srcm.at[pl.ds(nbase, 16)], sidx.at[1 - cur])
        pltpu.sync_copy(dstm.at[pl.ds(nbase, 16)], didx.at[1 - cur])
      scat[14].wait()
      scat[15].wait()
      return 0
    lax.fori_loop(0, ngroups, group, 0)

    plsc.subcore_barrier()
    pltpu.sync_copy(acc.at[pl.ds(rbase, _RB)],
                    out_hbm.at[cid, pl.ds(rbase, _RB)])
    @pl.when(sid == _NS - 1)
    def _():
      pltpu.sync_copy(acc.at[pl.ds(_NS * _RB, _N - _NS * _RB)],
                      out_hbm.at[cid, pl.ds(_NS * _RB, _N - _NS * _RB)])

  return pl.kernel(body, out_type=out_type, mesh=mesh, scratch_types=scratch)


def _make_sc_deg():
  mesh = plsc.VectorSubcoreMesh(core_axis_name="c", subcore_axis_name="s",
                                num_cores=_NC, num_subcores=_NS)
  out_type = [jax.ShapeDtypeStruct((_NC, _N, 128), jnp.float32)]
  scratch = [
      pltpu.VMEM((8, _CH), jnp.int32),
      pltpu.VMEM((_CH, 128), jnp.float32),
      pltpu.VMEM((_ZR, 128), jnp.float32),
      pltpu.VMEM_SHARED((_NACC, 128), jnp.float32),
  ]

  def body(dstm, deg_hbm, didx, ones, zdeg, degacc):
    cid = lax.axis_index("c")
    sid = lax.axis_index("s")
    wid = sid * _NC + cid
    def onesrow(i, _):
      def onelane(j, _):
        ones[i, pl.ds(j * 16, 16)] = jnp.ones((16,), jnp.float32)
        return 0
      return lax.fori_loop(0, 8, onelane, 0)
    lax.fori_loop(0, _CH, onesrow, 0)
    def zdrow(i, _):
      def zlane(j, _):
        zdeg[i, pl.ds(j * 16, 16)] = jnp.zeros((16,), jnp.float32)
        return 0
      return lax.fori_loop(0, 8, zlane, 0)
    lax.fori_loop(0, _ZR, zdrow, 0)
    rbase = sid * _RB
    def zdcopy(k, _):
      pltpu.sync_copy(zdeg, degacc.at[pl.ds(rbase + k * _ZR, _ZR)])
      return 0
    lax.fori_loop(0, _RB // _ZR, zdcopy, 0)
    @pl.when(sid == _NS - 1)
    def _():
      pltpu.sync_copy(zdeg, degacc.at[pl.ds(_NS * _RB, _N - _NS * _RB)])
    plsc.subcore_barrier()
    def group(k, _):
      gbase = pl.multiple_of((wid * _GPT + k) * 8, 8)
      pltpu.sync_copy(dstm.at[pl.ds(gbase, 8)], didx)
      for j in range(8):
        pltpu.sync_copy(ones, degacc.at[didx.at[j]], add=True)
      return 0
    lax.fori_loop(0, _GPT, group, 0)
    plsc.subcore_barrier()
    pltpu.sync_copy(degacc.at[pl.ds(rbase, _RB)],
                    deg_hbm.at[cid, pl.ds(rbase, _RB)])
    @pl.when(sid == _NS - 1)
    def _():
      pltpu.sync_copy(degacc.at[pl.ds(_NS * _RB, _N - _NS * _RB)],
                      deg_hbm.at[cid, pl.ds(_NS * _RB, _N - _NS * _RB)])

  return pl.kernel(body, out_type=out_type, mesh=mesh, scratch_types=scratch)


_AGG128 = _make_sc_agg(128)
_AGG128D = _make_sc_agg(128, with_deg=True)

_R = 1000
_EPAD = _EROWS * _CH


def _make_tc(pre, relu, din, da, dout, dnext):
  grid = (_N // _R,)
  row = lambda w: pl.BlockSpec((_R, w), lambda i: (i, 0))
  full = lambda a, b_: pl.BlockSpec((a, b_), lambda i: (0, 0))
  in_specs = [row(da), row(da), row(1), row(1), row(din)]
  if pre:
    in_specs.append(full(da, dout))
  in_specs.append(full(din, dout))
  in_specs.append(full(1, dout))
  if dnext:
    in_specs.append(full(dout, dnext))
  out_shape = [jax.ShapeDtypeStruct((_N, dout), jnp.float32)]
  out_specs = [row(dout)]
  if dnext:
    out_shape.append(jax.ShapeDtypeStruct((_N, dnext), jnp.float32))
    out_specs.append(row(dnext))

  def kfn(*refs):
    p0, p1, dg0, dg1, h = (r[...] for r in refs[:5])
    i = 5
    if pre:
      Wl = refs[i][...]; i += 1
    Wr = refs[i][...]; i += 1
    b = refs[i][...]; i += 1
    Wn = None
    if dnext:
      Wn = refs[i][...]; i += 1
    out_ref = refs[i]; i += 1
    gn_ref = refs[i] if dnext else None
    recip = 1.0 / jnp.maximum(dg0 + dg1, 1.0)
    mean = (p0 + p1) * recip
    if pre:
      acc = (jnp.dot(mean, Wl, preferred_element_type=jnp.float32)
             + jnp.dot(h, Wr, preferred_element_type=jnp.float32) + b)
    else:
      acc = mean + jnp.dot(h, Wr, preferred_element_type=jnp.float32) + b
    if relu:
      acc = jnp.maximum(acc, 0.0)
    out_ref[...] = acc
    if dnext:
      gn_ref[...] = jnp.dot(acc, Wn, preferred_element_type=jnp.float32)

  return pl.pallas_call(kfn, grid=grid, in_specs=in_specs,
                        out_specs=out_specs, out_shape=out_shape)


_TC1 = _make_tc(pre=True, relu=True, din=128, da=128, dout=256, dnext=128)
_TC2 = _make_tc(pre=False, relu=True, din=256, da=128, dout=128, dnext=128)
_TC3 = _make_tc(pre=False, relu=True, din=128, da=128, dout=128, dnext=None)
_TC4 = _make_tc(pre=True, relu=False, din=128, da=128, dout=128, dnext=None)


def kernel(x, edge_index, W1l, b1, W1r, W2l, b2, W2r, W3l, b3, W3r,
           W4l, b4, W4r):
  src = edge_index[0]
  dst = edge_index[1]
  pad_src = (jnp.arange(_EPAD - _E, dtype=jnp.int32) * 131) % _N
  srcm = jnp.concatenate([src, pad_src]).reshape(_EROWS, _CH)
  dstm = jnp.concatenate(
      [dst, jnp.full((_EPAD - _E,), _N, jnp.int32)]).reshape(_EROWS, _CH)
  W3l_p = jnp.pad(W3l, ((0, 0), (0, 64)))
  W3r_p = jnp.pad(W3r, ((0, 0), (0, 64)))
  b3_p = jnp.pad(b3, (0, 64))
  W4l_p = jnp.pad(W4l, ((0, 64), (0, 0)))
  W4r_p = jnp.pad(W4r, ((0, 64), (0, 0)))

  p1, degs = _AGG128D(x, srcm, dstm)
  dg0 = degs[0, :, 0:1]
  dg1 = degs[1, :, 0:1]

  h1, g2 = _TC1(p1[0], p1[1], dg0, dg1, x, W1l, W1r, b1.reshape(1, -1), W2l)
  (p2,) = _AGG128(g2, srcm, dstm)
  h2, g3 = _TC2(p2[0], p2[1], dg0, dg1, h1, W2r, b2.reshape(1, -1), W3l_p)
  (p3,) = _AGG128(g3, srcm, dstm)
  (h3,) = _TC3(p3[0], p3[1], dg0, dg1, h2, W3r_p, b3_p.reshape(1, -1))
  (p4,) = _AGG128(h3, srcm, dstm)
  (out,) = _TC4(p4[0], p4[1], dg0, dg1, h3, W4l_p, W4r_p, b4.reshape(1, -1))
  return out


# speedup vs baseline: 3.3202x; 1.0001x over previous
import jax
import jax.numpy as jnp
from jax import lax
from jax.experimental import pallas as pl
from jax.experimental.pallas import tpu as pltpu
from jax.experimental.pallas import tpu_sc as plsc

_N = 10000
_E = 320000
_NC = 2
_NS = 16
_NW = _NC * _NS
_CH = 128
_EROWS = 2560
_GPT = _EROWS // _NW // 8
_RB = 624
_ZR = 16
_DW = 16
_NACC = _N + 16


_R0 = 80
_R1 = 160 - _R0
_C1BASE = _NS * _R0


def _make_sc_agg(d, with_deg=False):
  mesh = plsc.VectorSubcoreMesh(core_axis_name="c", subcore_axis_name="s",
                                num_cores=_NC, num_subcores=_NS)
  out_type = [jax.ShapeDtypeStruct((_NC, _N, d), jnp.float32)]
  if with_deg:
    out_type.append(jax.ShapeDtypeStruct((_NC, _N, d), jnp.float32))
  scratch = [
      pltpu.VMEM((2, 16, _CH), jnp.int32),
      pltpu.VMEM((2, 16, _CH), jnp.int32),
      pltpu.VMEM((2, _CH, d), jnp.float32),
      pltpu.VMEM((_ZR, d), jnp.float32),
      pltpu.VMEM_SHARED((_NACC, d), jnp.float32),
      pltpu.SemaphoreType.DMA,
      pltpu.SemaphoreType.DMA,
  ]

  def body(g_hbm, srcm, dstm, out_hbm, *rest):
    if with_deg:
      deg_hbm, sidx, didx, rows2, zbuf, acc, semg, sems = rest
    else:
      sidx, didx, rows2, zbuf, acc, semg, sems = rest
    cid = lax.axis_index("c")
    sid = lax.axis_index("s")
    tbase = jnp.where(cid == 0, sid * _R0, _C1BASE + sid * _R1)
    ngroups = jnp.where(cid == 0, _R0 // 16, _R1 // 16)

    g0 = pl.multiple_of(tbase + 0, 8)
    pltpu.sync_copy(srcm.at[pl.ds(g0, 16)], sidx.at[0])
    pltpu.sync_copy(dstm.at[pl.ds(g0, 16)], didx.at[0])

    def zrow(i, _):
      def zlane(j, _):
        zbuf[i, pl.ds(j * 16, 16)] = jnp.zeros((16,), jnp.float32)
        return 0
      return lax.fori_loop(0, d // 16, zlane, 0)
    lax.fori_loop(0, _ZR, zrow, 0)
    rbase = sid * _RB
    def zcopy(k, _):
      pltpu.sync_copy(zbuf, acc.at[pl.ds(rbase + k * _ZR, _ZR)])
      return 0
    lax.fori_loop(0, _RB // _ZR, zcopy, 0)
    @pl.when(sid == _NS - 1)
    def _():
      pltpu.sync_copy(zbuf, acc.at[pl.ds(_NS * _RB, _N - _NS * _RB)])

    plsc.subcore_barrier()

    if with_deg:
      def onesrow(i, _):
        def onelane(j, _):
          rows2[0, i, pl.ds(j * 16, 16)] = jnp.ones((16,), jnp.float32)
          return 0
        return lax.fori_loop(0, d // 16, onelane, 0)
      lax.fori_loop(0, _CH, onesrow, 0)

      def dgroup(k, _):
        cur = k % 2
        for j in range(16):
          pltpu.sync_copy(rows2.at[0], acc.at[didx.at[cur, j]], add=True)
        @pl.when(k + 1 < ngroups)
        def _():
          nbase = pl.multiple_of(tbase + (k + 1) * 16, 8)
          pltpu.sync_copy(dstm.at[pl.ds(nbase, 16)], didx.at[1 - cur])
        return 0
      lax.fori_loop(0, ngroups, dgroup, 0)

      plsc.subcore_barrier()
      pltpu.sync_copy(acc.at[pl.ds(rbase, _RB)],
                      deg_hbm.at[cid, pl.ds(rbase, _RB)])
      @pl.when(sid == _NS - 1)
      def _():
        pltpu.sync_copy(acc.at[pl.ds(_NS * _RB, _N - _NS * _RB)],
                        deg_hbm.at[cid, pl.ds(_NS * _RB, _N - _NS * _RB)])
      def rz(i, _):
        def rzl(j, _):
          rows2[0, i, pl.ds(j * 16, 16)] = jnp.zeros((16,), jnp.float32)
          return 0
        return lax.fori_loop(0, d // 16, rzl, 0)
      lax.fori_loop(0, _ZR, rz, 0)
      def rzcopy(k, _):
        pltpu.sync_copy(rows2.at[0, pl.ds(0, _ZR)],
                        acc.at[pl.ds(rbase + k * _ZR, _ZR)])
        return 0
      lax.fori_loop(0, _RB // _ZR, rzcopy, 0)
      @pl.when(sid == _NS - 1)
      def _():
        pltpu.sync_copy(rows2.at[0, pl.ds(0, _ZR)],
                        acc.at[pl.ds(_NS * _RB, _N - _NS * _RB)])
      pltpu.sync_copy(dstm.at[pl.ds(g0, 16)], didx.at[0])
      plsc.subcore_barrier()

    def group(k, _):
      cur = k % 2
      scat = {}
      gat = pltpu.async_copy(g_hbm.at[sidx.at[cur, 0]], rows2.at[0], semg)
      for j in range(16):
        gat.wait()
        scat[j] = pltpu.async_copy(rows2.at[j % 2], acc.at[didx.at[cur, j]],
                                   sems, add=True)
        if j < 15:
          if j >= 1:
            scat[j - 1].wait()
          gat = pltpu.async_copy(g_hbm.at[sidx.at[cur, j + 1]],
                                 rows2.at[(j + 1) % 2], semg)
      @pl.when(k + 1 < ngroups)
      def _():
        nbase = pl.multiple_of(tbase + (k + 1) * 16, 8)
        pltpu.sync_copy(srcm.at[pl.ds(nbase, 16)], sidx.at[1 - cur])
        pltpu.sync_copy(dstm.at[pl.ds(nbase, 16)], didx.at[1 - cur])
      scat[14].wait()
      scat[15].wait()
      return 0
    lax.fori_loop(0, ngroups, group, 0)

    plsc.subcore_barrier()
    pltpu.sync_copy(acc.at[pl.ds(rbase, _RB)],
                    out_hbm.at[cid, pl.ds(rbase, _RB)])
    @pl.when(sid == _NS - 1)
    def _():
      pltpu.sync_copy(acc.at[pl.ds(_NS * _RB, _N - _NS * _RB)],
                      out_hbm.at[cid, pl.ds(_NS * _RB, _N - _NS * _RB)])

  return pl.kernel(body, out_type=out_type, mesh=mesh, scratch_types=scratch)


_AGG128 = _make_sc_agg(128)
_AGG128D = _make_sc_agg(128, with_deg=True)

_R = 1000
_EPAD = _EROWS * _CH


def _make_tc(pre, relu, din, da, dout, dnext):
  grid = (_N // _R,)
  row = lambda w: pl.BlockSpec((_R, w), lambda i: (i, 0))
  full = lambda a, b_: pl.BlockSpec((a, b_), lambda i: (0, 0))
  in_specs = [row(da), row(da), row(1), row(1), row(din)]
  if pre:
    in_specs.append(full(da, dout))
  in_specs.append(full(din, dout))
  in_specs.append(full(1, dout))
  if dnext:
    in_specs.append(full(dout, dnext))
  out_shape = [jax.ShapeDtypeStruct((_N, dout), jnp.float32)]
  out_specs = [row(dout)]
  if dnext:
    out_shape.append(jax.ShapeDtypeStruct((_N, dnext), jnp.float32))
    out_specs.append(row(dnext))

  def kfn(*refs):
    p0, p1, dg0, dg1, h = (r[...] for r in refs[:5])
    i = 5
    if pre:
      Wl = refs[i][...]; i += 1
    Wr = refs[i][...]; i += 1
    b = refs[i][...]; i += 1
    Wn = None
    if dnext:
      Wn = refs[i][...]; i += 1
    out_ref = refs[i]; i += 1
    gn_ref = refs[i] if dnext else None
    recip = 1.0 / jnp.maximum(dg0 + dg1, 1.0)
    mean = (p0 + p1) * recip
    if pre:
      acc = (jnp.dot(mean, Wl, preferred_element_type=jnp.float32)
             + jnp.dot(h, Wr, preferred_element_type=jnp.float32) + b)
    else:
      acc = mean + jnp.dot(h, Wr, preferred_element_type=jnp.float32) + b
    if relu:
      acc = jnp.maximum(acc, 0.0)
    out_ref[...] = acc
    if dnext:
      gn_ref[...] = jnp.dot(acc, Wn, preferred_element_type=jnp.float32)

  return pl.pallas_call(kfn, grid=grid, in_specs=in_specs,
                        out_specs=out_specs, out_shape=out_shape)


_TC1 = _make_tc(pre=True, relu=True, din=128, da=128, dout=256, dnext=128)
_TC2 = _make_tc(pre=False, relu=True, din=256, da=128, dout=128, dnext=128)
_TC3 = _make_tc(pre=False, relu=True, din=128, da=128, dout=128, dnext=None)
_TC4 = _make_tc(pre=True, relu=False, din=128, da=128, dout=128, dnext=None)


def kernel(x, edge_index, W1l, b1, W1r, W2l, b2, W2r, W3l, b3, W3r,
           W4l, b4, W4r):
  src = edge_index[0]
  dst = edge_index[1]
  pad_src = (jnp.arange(_EPAD - _E, dtype=jnp.int32) * 131) % _N
  srcm = jnp.concatenate([src, pad_src]).reshape(_EROWS, _CH)
  dstm = jnp.concatenate(
      [dst, jnp.full((_EPAD - _E,), _N, jnp.int32)]).reshape(_EROWS, _CH)
  W3l_p = jnp.pad(W3l, ((0, 0), (0, 64)))
  W3r_p = jnp.pad(W3r, ((0, 0), (0, 64)))
  b3_p = jnp.pad(b3, (0, 64))
  W4l_p = jnp.pad(W4l, ((0, 64), (0, 0)))
  W4r_p = jnp.pad(W4r, ((0, 64), (0, 0)))

  p1, degs = _AGG128D(x, srcm, dstm)
  dg0 = degs[0, :, 0:1]
  dg1 = degs[1, :, 0:1]

  h1, g2 = _TC1(p1[0], p1[1], dg0, dg1, x, W1l, W1r, b1.reshape(1, -1), W2l)
  (p2,) = _AGG128(g2, srcm, dstm)
  h2, g3 = _TC2(p2[0], p2[1], dg0, dg1, h1, W2r, b2.reshape(1, -1), W3l_p)
  (p3,) = _AGG128(g3, srcm, dstm)
  (h3,) = _TC3(p3[0], p3[1], dg0, dg1, h2, W3r_p, b3_p.reshape(1, -1))
  (p4,) = _AGG128(h3, srcm, dstm)
  (out,) = _TC4(p4[0], p4[1], dg0, dg1, h3, W4l_p, W4r_p, b4.reshape(1, -1))
  return out
